# baseline (speedup 1.0000x reference)
import functools

import jax
import jax.numpy as jnp
from jax import lax
from jax.experimental import pallas as pl
from jax.experimental.pallas import tpu as pltpu
from jax.experimental.pallas import tpu_sc as plsc

_NEG_INF = float("-inf")
_I32_MAX = 2**31 - 1


def _reduce_body(V, C, NBLK, x_ref, logz_ref, mode_ref, m_sc, s_sc, bv_sc, bi_sc):
    j = pl.program_id(0)
    B = x_ref.shape[0]
    col = lax.broadcasted_iota(jnp.int32, (B, C), 1)

    def _stats(masked):
        x = x_ref[...]
        if masked:
            x = jnp.where(j * C + col < V, x, _NEG_INF)
        bmax = jnp.max(x, axis=-1, keepdims=True)
        bidx = jnp.min(jnp.where(x == bmax, col, _I32_MAX), axis=-1, keepdims=True)
        bsum = jnp.sum(jnp.exp(x - bmax), axis=-1, keepdims=True)
        return bmax, bidx + j * C, bsum

    def _acc(bmax, bidx, bsum):
        m_old = m_sc[...]
        m_new = jnp.maximum(m_old, bmax)
        s_sc[...] = s_sc[...] * jnp.exp(m_old - m_new) + bsum * jnp.exp(bmax - m_new)
        m_sc[...] = m_new
        better = bmax > bv_sc[...]
        bv_sc[...] = jnp.where(better, bmax, bv_sc[...])
        bi_sc[...] = jnp.where(better, bidx, bi_sc[...])

    tail_masked = V % C != 0

    @pl.when(j == 0)
    def _init():
        bmax, bidx, bsum = _stats(masked=tail_masked and NBLK == 1)
        m_sc[...] = bmax
        s_sc[...] = bsum
        bv_sc[...] = bmax
        bi_sc[...] = bidx

    @pl.when((j > 0) & (j < NBLK - 1))
    def _mid():
        _acc(*_stats(masked=False))

    @pl.when((j == NBLK - 1) & (j > 0))
    def _last():
        _acc(*_stats(masked=tail_masked))

    @pl.when(j == NBLK - 1)
    def _fin():
        logz_ref[...] = m_sc[...] + jnp.log(s_sc[...])
        mode_ref[...] = bi_sc[...]


def _fused_pass(logits, C=16384):
    B, V = logits.shape
    NBLK = pl.cdiv(V, C)
    return pl.pallas_call(
        functools.partial(_reduce_body, V, C, NBLK),
        grid=(NBLK,),
        in_specs=[pl.BlockSpec((B, C), lambda j: (0, j))],
        out_specs=[pl.BlockSpec((B, 1), lambda j: (0, 0)),
                   pl.BlockSpec((B, 1), lambda j: (0, 0))],
        out_shape=[jax.ShapeDtypeStruct((B, 1), jnp.float32),
                   jax.ShapeDtypeStruct((B, 1), jnp.int32)],
        scratch_shapes=[pltpu.VMEM((B, 1), jnp.float32),
                        pltpu.VMEM((B, 1), jnp.float32),
                        pltpu.VMEM((B, 1), jnp.float32),
                        pltpu.VMEM((B, 1), jnp.int32)],
    )(logits)


def _sc_gather(flat, idx):
    info = plsc.get_sparse_core_info()
    nc, L = info.num_cores, info.num_lanes
    B = idx.shape[0]
    nw = B // L
    mesh = plsc.VectorSubcoreMesh(core_axis_name="c", subcore_axis_name="s")

    @functools.partial(
        pl.kernel, mesh=mesh,
        out_type=jax.ShapeDtypeStruct((B,), jnp.float32),
        scratch_types=[pltpu.VMEM((L,), jnp.int32),
                       pltpu.VMEM((L,), jnp.float32),
                       pltpu.SemaphoreType.DMA],
    )
    def k(flat_hbm, idx_hbm, out_hbm, idx_v, vals_v, sem):
        wid = lax.axis_index("s") * nc + lax.axis_index("c")

        @pl.when(wid < nw)
        def _():
            base = wid * L
            pltpu.sync_copy(idx_hbm.at[pl.ds(base, L)], idx_v)
            pltpu.async_copy(flat_hbm.at[idx_v], vals_v, sem).wait()
            pltpu.sync_copy(vals_v, out_hbm.at[pl.ds(base, L)])

    return k(flat, idx)


def kernel(logits, actions):
    B, V = logits.shape
    a = actions.reshape(B).astype(jnp.int32)
    flat_idx = jnp.arange(B, dtype=jnp.int32) * V + a
    gathered = _sc_gather(logits.reshape(B * V), flat_idx)
    logz, mode = _fused_pass(logits)
    log_probs = gathered[:, None] - logz
    return log_probs, mode


# speedup vs baseline: 37.4466x; 37.4466x over previous
import functools

import jax
import jax.numpy as jnp
from jax import lax
from jax.experimental import pallas as pl
from jax.experimental.pallas import tpu as pltpu
from jax.experimental.pallas import tpu_sc as plsc

_NEG_INF = float("-inf")
_I32_MAX = 2**31 - 1


def _reduce_body(V, C, NBLK, x_ref, logz_ref, mode_ref, m_sc, s_sc, bv_sc, bi_sc):
    j = pl.program_id(0)
    B = x_ref.shape[0]
    col = lax.broadcasted_iota(jnp.int32, (B, C), 1)

    def _stats(masked):
        x = x_ref[...]
        if masked:
            x = jnp.where(j * C + col < V, x, _NEG_INF)
        bmax = jnp.max(x, axis=-1, keepdims=True)
        bidx = jnp.min(jnp.where(x == bmax, col, _I32_MAX), axis=-1, keepdims=True)
        bsum = jnp.sum(jnp.exp(x - bmax), axis=-1, keepdims=True)
        return bmax, bidx + j * C, bsum

    def _acc(bmax, bidx, bsum):
        m_old = m_sc[...]
        m_new = jnp.maximum(m_old, bmax)
        s_sc[...] = s_sc[...] * jnp.exp(m_old - m_new) + bsum * jnp.exp(bmax - m_new)
        m_sc[...] = m_new
        better = bmax > bv_sc[...]
        bv_sc[...] = jnp.where(better, bmax, bv_sc[...])
        bi_sc[...] = jnp.where(better, bidx, bi_sc[...])

    tail_masked = V % C != 0

    @pl.when(j == 0)
    def _init():
        bmax, bidx, bsum = _stats(masked=tail_masked and NBLK == 1)
        m_sc[...] = bmax
        s_sc[...] = bsum
        bv_sc[...] = bmax
        bi_sc[...] = bidx

    @pl.when((j > 0) & (j < NBLK - 1))
    def _mid():
        _acc(*_stats(masked=False))

    @pl.when((j == NBLK - 1) & (j > 0))
    def _last():
        _acc(*_stats(masked=tail_masked))

    @pl.when(j == NBLK - 1)
    def _fin():
        logz_ref[...] = m_sc[...] + jnp.log(s_sc[...])
        mode_ref[...] = bi_sc[...]


def _fused_pass(logits, C=16384):
    B, V = logits.shape
    NBLK = pl.cdiv(V, C)
    return pl.pallas_call(
        functools.partial(_reduce_body, V, C, NBLK),
        grid=(NBLK,),
        in_specs=[pl.BlockSpec((B, C), lambda j: (0, j))],
        out_specs=[pl.BlockSpec((B, 1), lambda j: (0, 0)),
                   pl.BlockSpec((B, 1), lambda j: (0, 0))],
        out_shape=[jax.ShapeDtypeStruct((B, 1), jnp.float32),
                   jax.ShapeDtypeStruct((B, 1), jnp.int32)],
        scratch_shapes=[pltpu.VMEM((B, 1), jnp.float32),
                        pltpu.VMEM((B, 1), jnp.float32),
                        pltpu.VMEM((B, 1), jnp.float32),
                        pltpu.VMEM((B, 1), jnp.int32)],
    )(logits)


def _sc_gather(logits, idx):
    info = plsc.get_sparse_core_info()
    nc, L = info.num_cores, info.num_lanes
    B, V = logits.shape
    nw = B // L
    mesh = plsc.VectorSubcoreMesh(core_axis_name="c", subcore_axis_name="s")

    @functools.partial(
        pl.kernel, mesh=mesh,
        out_type=jax.ShapeDtypeStruct((B, 128), jnp.float32),
        scratch_types=[pltpu.VMEM((L,), jnp.int32),
                       pltpu.VMEM((L, 8, 128), jnp.float32),
                       pltpu.VMEM((L, 128), jnp.float32)],
    )
    def k(x_hbm, idx_hbm, out_hbm, idx_v, buf_v, seg_v):
        wid = lax.axis_index("s") * nc + lax.axis_index("c")

        @pl.when(wid < nw)
        def _():
            row0 = wid * L
            pltpu.sync_copy(idx_hbm.at[pl.ds(row0, L)], idx_v)
            idx_reg = idx_v[...]
            for i in range(L):
                base = pl.multiple_of(lax.bitwise_and(idx_reg[i], -128), 128)
                r0 = pl.multiple_of(row0 + (i & ~7), 8)
                pltpu.sync_copy(x_hbm.at[pl.ds(r0, 8), pl.ds(base, 128)],
                                buf_v.at[i])
                for k16 in range(8):
                    seg_v[i, pl.ds(16 * k16, 16)] = buf_v[i, i & 7, pl.ds(16 * k16, 16)]
            pltpu.sync_copy(
                seg_v, out_hbm.at[pl.ds(pl.multiple_of(row0, 8), L), :])

    return k(logits, idx)


def _combine_body(a_ref, seg_ref, logz_ref, lp_ref):
    B = a_ref.shape[0]
    c = lax.bitwise_and(a_ref[...], 127)
    col = lax.broadcasted_iota(jnp.int32, (B, 128), 1)
    g = jnp.sum(jnp.where(col == c, seg_ref[...], 0.0), axis=-1, keepdims=True)
    lp_ref[...] = g - logz_ref[...]


def _combine(actions, seg, logz):
    B = actions.shape[0]
    return pl.pallas_call(
        _combine_body,
        out_shape=jax.ShapeDtypeStruct((B, 1), jnp.float32),
    )(actions, seg, logz)


def kernel(logits, actions):
    B, V = logits.shape
    a = actions.reshape(B).astype(jnp.int32)
    seg = _sc_gather(logits, a)
    logz, mode = _fused_pass(logits)
    log_probs = _combine(actions.astype(jnp.int32), seg, logz)
    return log_probs, mode


# speedup vs baseline: 40.9964x; 1.0948x over previous
import functools

import jax
import jax.numpy as jnp
from jax import lax
from jax.experimental import pallas as pl
from jax.experimental.pallas import tpu as pltpu
from jax.experimental.pallas import tpu_sc as plsc

_NEG_INF = float("-inf")
_I32_MAX = 2**31 - 1


def _reduce_body(V, C, NBLK, x_ref, logz_ref, mode_ref, m_sc, s_sc, bv_sc, bi_sc):
    j = pl.program_id(0)
    B = x_ref.shape[0]
    col = lax.broadcasted_iota(jnp.int32, (B, C), 1)

    def _stats(masked):
        x = x_ref[...]
        if masked:
            x = jnp.where(j * C + col < V, x, _NEG_INF)
        bmax = jnp.max(x, axis=-1, keepdims=True)
        bidx = jnp.min(jnp.where(x == bmax, col, _I32_MAX), axis=-1, keepdims=True)
        bsum = jnp.sum(jnp.exp(x - bmax), axis=-1, keepdims=True)
        return bmax, bidx + j * C, bsum

    def _acc(bmax, bidx, bsum):
        m_old = m_sc[...]
        m_new = jnp.maximum(m_old, bmax)
        s_sc[...] = s_sc[...] * jnp.exp(m_old - m_new) + bsum * jnp.exp(bmax - m_new)
        m_sc[...] = m_new
        better = bmax > bv_sc[...]
        bv_sc[...] = jnp.where(better, bmax, bv_sc[...])
        bi_sc[...] = jnp.where(better, bidx, bi_sc[...])

    tail_masked = V % C != 0

    @pl.when(j == 0)
    def _init():
        bmax, bidx, bsum = _stats(masked=tail_masked and NBLK == 1)
        m_sc[...] = bmax
        s_sc[...] = bsum
        bv_sc[...] = bmax
        bi_sc[...] = bidx

    @pl.when((j > 0) & (j < NBLK - 1))
    def _mid():
        _acc(*_stats(masked=False))

    @pl.when((j == NBLK - 1) & (j > 0))
    def _last():
        _acc(*_stats(masked=tail_masked))

    @pl.when(j == NBLK - 1)
    def _fin():
        logz_ref[...] = m_sc[...] + jnp.log(s_sc[...])
        mode_ref[...] = bi_sc[...]


def _fused_pass(logits, C=32768):
    B, V = logits.shape
    NBLK = pl.cdiv(V, C)
    return pl.pallas_call(
        functools.partial(_reduce_body, V, C, NBLK),
        grid=(NBLK,),
        in_specs=[pl.BlockSpec((B, C), lambda j: (0, j))],
        out_specs=[pl.BlockSpec((B, 1), lambda j: (0, 0)),
                   pl.BlockSpec((B, 1), lambda j: (0, 0))],
        out_shape=[jax.ShapeDtypeStruct((B, 1), jnp.float32),
                   jax.ShapeDtypeStruct((B, 1), jnp.int32)],
        scratch_shapes=[pltpu.VMEM((B, 1), jnp.float32),
                        pltpu.VMEM((B, 1), jnp.float32),
                        pltpu.VMEM((B, 1), jnp.float32),
                        pltpu.VMEM((B, 1), jnp.int32)],
    )(logits)


def _sc_gather(logits, idx):
    info = plsc.get_sparse_core_info()
    nc, L = info.num_cores, info.num_lanes
    B, V = logits.shape
    nw = B // L
    mesh = plsc.VectorSubcoreMesh(core_axis_name="c", subcore_axis_name="s")

    @functools.partial(
        pl.kernel, mesh=mesh,
        out_type=jax.ShapeDtypeStruct((B, 128), jnp.float32),
        scratch_types=[pltpu.VMEM((L,), jnp.int32),
                       pltpu.VMEM((L, 8, 128), jnp.float32),
                       pltpu.VMEM((L, 128), jnp.float32)],
    )
    def k(x_hbm, idx_hbm, out_hbm, idx_v, buf_v, seg_v):
        wid = lax.axis_index("s") * nc + lax.axis_index("c")

        @pl.when(wid < nw)
        def _():
            row0 = wid * L
            pltpu.sync_copy(idx_hbm.at[pl.ds(row0, L)], idx_v)
            idx_reg = idx_v[...]
            for i in range(L):
                base = pl.multiple_of(lax.bitwise_and(idx_reg[i], -128), 128)
                r0 = pl.multiple_of(row0 + (i & ~7), 8)
                pltpu.sync_copy(x_hbm.at[pl.ds(r0, 8), pl.ds(base, 128)],
                                buf_v.at[i])
                for k16 in range(8):
                    seg_v[i, pl.ds(16 * k16, 16)] = buf_v[i, i & 7, pl.ds(16 * k16, 16)]
            pltpu.sync_copy(
                seg_v, out_hbm.at[pl.ds(pl.multiple_of(row0, 8), L), :])

    return k(logits, idx)


def _combine_body(a_ref, seg_ref, logz_ref, lp_ref):
    B = a_ref.shape[0]
    c = lax.bitwise_and(a_ref[...], 127)
    col = lax.broadcasted_iota(jnp.int32, (B, 128), 1)
    g = jnp.sum(jnp.where(col == c, seg_ref[...], 0.0), axis=-1, keepdims=True)
    lp_ref[...] = g - logz_ref[...]


def _combine(actions, seg, logz):
    B = actions.shape[0]
    return pl.pallas_call(
        _combine_body,
        out_shape=jax.ShapeDtypeStruct((B, 1), jnp.float32),
    )(actions, seg, logz)


def kernel(logits, actions):
    B, V = logits.shape
    a = actions.reshape(B).astype(jnp.int32)
    seg = _sc_gather(logits, a)
    logz, mode = _fused_pass(logits)
    log_probs = _combine(actions.astype(jnp.int32), seg, logz)
    return log_probs, mode
